# baseline (speedup 1.0000x reference)
import functools

import jax
import jax.numpy as jnp
from jax import lax
from jax.experimental import pallas as pl
from jax.experimental.pallas import tpu as pltpu
from jax.experimental.pallas import tpu_sc as plsc

B = 16384
FACT = 64
IMGF = 512

_info = plsc.get_sparse_core_info()
NC = _info.num_cores
NS = _info.num_subcores
NW = NC * NS
BPW = B // NW
CH = 64
NCH = BPW // CH

_MESH = plsc.VectorSubcoreMesh(core_axis_name="c", subcore_axis_name="s")


def _sc_gather_f(item2, F):
    @functools.partial(
        pl.kernel, mesh=_MESH,
        out_type=jax.ShapeDtypeStruct((B, IMGF), jnp.float32),
        scratch_types=[
            pltpu.VMEM((NCH, CH), jnp.int32),
            pltpu.VMEM((2, CH, IMGF), jnp.float32),
            pltpu.SemaphoreType.DMA,
            pltpu.SemaphoreType.DMA,
        ],
    )
    def k(item_h, f_h, fi_o, iidx, fb, gsem, wsem):
        wid = lax.axis_index("s") * NC + lax.axis_index("c")
        base = wid * BPW
        pltpu.sync_copy(item_h.at[wid], iidx)
        h_g = [None, None]
        h_w = [None, None]
        for c in range(NCH):
            s = c % 2
            if h_w[s] is not None:
                h_w[s].wait()
            h_g[s] = pltpu.async_copy(f_h.at[iidx.at[c]], fb.at[s], gsem)
            if c > 0:
                p = 1 - s
                h_g[p].wait()
                h_w[p] = pltpu.async_copy(
                    fb.at[p], fi_o.at[pl.ds(base + (c - 1) * CH, CH)], wsem)
        last = (NCH - 1) % 2
        h_g[last].wait()
        h_w[last] = pltpu.async_copy(
            fb.at[last], fi_o.at[pl.ds(base + (NCH - 1) * CH, CH)], wsem)
        h_w[1 - last].wait()
        h_w[last].wait()

    return k(item2, F)


def _sc_gather_small(user2, item2, Gu16, Gi16, L16):
    @functools.partial(
        pl.kernel, mesh=_MESH,
        out_type=[
            jax.ShapeDtypeStruct((B, FACT), jnp.bfloat16),
            jax.ShapeDtypeStruct((B, FACT), jnp.bfloat16),
            jax.ShapeDtypeStruct((B, FACT), jnp.bfloat16),
        ],
        scratch_types=[
            pltpu.VMEM((NCH, CH), jnp.int32),
            pltpu.VMEM((NCH, CH), jnp.int32),
            pltpu.VMEM((2, CH, FACT), jnp.bfloat16),
            pltpu.VMEM((2, CH, FACT), jnp.bfloat16),
            pltpu.VMEM((2, CH, FACT), jnp.bfloat16),
            pltpu.SemaphoreType.DMA,
            pltpu.SemaphoreType.DMA,
        ],
        compiler_params=pltpu.CompilerParams(use_tc_tiling_on_sc=False),
    )
    def k(user_h, item_h, gu_h, gi_h, l_h,
          gu_o, gi_o, li_o, uidx, iidx, gub, gib, lb, gsem, wsem):
        wid = lax.axis_index("s") * NC + lax.axis_index("c")
        base = wid * BPW
        pltpu.sync_copy(user_h.at[wid], uidx)
        pltpu.sync_copy(item_h.at[wid], iidx)
        bufs = (gub, gib, lb)
        h_g = [None, None]
        h_w = [None, None]
        for c in range(NCH):
            s = c % 2
            if h_w[s] is not None:
                for h in h_w[s]:
                    h.wait()
            h_g[s] = (
                pltpu.async_copy(gu_h.at[uidx.at[c]], gub.at[s], gsem),
                pltpu.async_copy(gi_h.at[iidx.at[c]], gib.at[s], gsem),
                pltpu.async_copy(l_h.at[iidx.at[c]], lb.at[s], gsem),
            )
            if c > 0:
                p = 1 - s
                off = base + (c - 1) * CH
                for h in h_g[p]:
                    h.wait()
                h_w[p] = tuple(
                    pltpu.async_copy(bf.at[p], o.at[pl.ds(off, CH)], wsem)
                    for bf, o in zip(bufs, (gu_o, gi_o, li_o)))
        last = (NCH - 1) % 2
        off = base + (NCH - 1) * CH
        for h in h_g[last]:
            h.wait()
        h_w[last] = tuple(
            pltpu.async_copy(bf.at[last], o.at[pl.ds(off, CH)], wsem)
            for bf, o in zip(bufs, (gu_o, gi_o, li_o)))
        for h in h_w[1 - last]:
            h.wait()
        for h in h_w[last]:
            h.wait()

    return k(user2, item2, Gu16, Gi16, L16)


def _tc_combine(feature_i, gu16, gi16, l16, E):
    BB = 2048

    def body(fe, gu_r, gi_r, li_r, e, xout, guo, gio, lio):
        gu = gu_r[...].astype(jnp.float32)
        gi = gi_r[...].astype(jnp.float32)
        li = li_r[...].astype(jnp.float32)
        t = jnp.dot(fe[...], e[...], preferred_element_type=jnp.float32)
        xout[...] = jnp.sum(gu * (t - li + gi), axis=1, keepdims=True)
        guo[...] = gu
        gio[...] = gi
        lio[...] = li

    spec64 = pl.BlockSpec((BB, FACT), lambda i: (i, 0))
    return pl.pallas_call(
        body,
        grid=(B // BB,),
        in_specs=[
            pl.BlockSpec((BB, IMGF), lambda i: (i, 0)),
            spec64, spec64, spec64,
            pl.BlockSpec((IMGF, FACT), lambda i: (0, 0)),
        ],
        out_specs=[
            pl.BlockSpec((BB, 1), lambda i: (i, 0)),
            spec64, spec64, spec64,
        ],
        out_shape=[
            jax.ShapeDtypeStruct((B, 1), jnp.float32),
            jax.ShapeDtypeStruct((B, FACT), jnp.float32),
            jax.ShapeDtypeStruct((B, FACT), jnp.float32),
            jax.ShapeDtypeStruct((B, FACT), jnp.float32),
        ],
    )(feature_i, gu16, gi16, l16, E)


def kernel(user, item, Gu, Gi, L, E, F):
    user2 = user.reshape(NW, NCH, CH)
    item2 = item.reshape(NW, NCH, CH)
    Gu16 = Gu.astype(jnp.bfloat16)
    Gi16 = Gi.astype(jnp.bfloat16)
    L16 = L.astype(jnp.bfloat16)
    feature_i = _sc_gather_f(item2, F)
    gu16, gi16, l16 = _sc_gather_small(user2, item2, Gu16, Gi16, L16)
    xui, gamma_u, gamma_i, l_i = _tc_combine(feature_i, gu16, gi16, l16, E)
    return (xui.reshape(B), gamma_u, gamma_i, feature_i, l_i)


# speedup vs baseline: 1.9014x; 1.9014x over previous
import functools

import jax
import jax.numpy as jnp
from jax import lax
from jax.experimental import pallas as pl
from jax.experimental.pallas import tpu as pltpu
from jax.experimental.pallas import tpu_sc as plsc

B = 16384
FACT = 64
IMGF = 512
NU = 1000000
NI = 100000

_info = plsc.get_sparse_core_info()
NC = _info.num_cores
NS = _info.num_subcores
NW = NC * NS
BPW = B // NW
CH = 64
NCH = BPW // CH

_MESH = plsc.VectorSubcoreMesh(core_axis_name="c", subcore_axis_name="s")


BBC = 2048
HPU = 245 * BBC
HPI = 25 * BBC


def _tc_pair_transpose(Tt, nb):
    N = Tt.shape[1]
    nvb = (N + BBC - 1) // BBC - 1

    def body(a, b, o):
        eye = jnp.eye(FACT, dtype=jnp.float32)
        dn = (((0,), (0,)), ((), ()))
        o[:, 0:FACT] = lax.dot_general(
            a[...], eye, dn, preferred_element_type=jnp.float32)
        o[:, FACT:2 * FACT] = lax.dot_general(
            b[...], eye, dn, preferred_element_type=jnp.float32)

    return pl.pallas_call(
        body,
        grid=(nb,),
        in_specs=[
            pl.BlockSpec((FACT, BBC), lambda i: (0, i)),
            pl.BlockSpec((FACT, BBC), lambda i: (0, jnp.minimum(i + nb, nvb))),
        ],
        out_specs=pl.BlockSpec((BBC, 2 * FACT), lambda i: (i, 0)),
        out_shape=jax.ShapeDtypeStruct((nb * BBC, 2 * FACT), jnp.float32),
    )(Tt, Tt)


def _sc_gather_f(item2, F):
    @functools.partial(
        pl.kernel, mesh=_MESH,
        out_type=jax.ShapeDtypeStruct((B, IMGF), jnp.float32),
        scratch_types=[
            pltpu.VMEM((NCH, CH), jnp.int32),
            pltpu.VMEM((2, CH, IMGF), jnp.float32),
            pltpu.SemaphoreType.DMA,
            pltpu.SemaphoreType.DMA,
        ],
    )
    def k(item_h, f_h, fi_o, iidx, fb, gsem, wsem):
        wid = lax.axis_index("s") * NC + lax.axis_index("c")
        base = wid * BPW
        pltpu.sync_copy(item_h.at[wid], iidx)
        h_g = [None, None]
        h_w = [None, None]
        for c in range(NCH):
            s = c % 2
            if h_w[s] is not None:
                h_w[s].wait()
            h_g[s] = pltpu.async_copy(f_h.at[iidx.at[c]], fb.at[s], gsem)
            if c > 0:
                p = 1 - s
                h_g[p].wait()
                h_w[p] = pltpu.async_copy(
                    fb.at[p], fi_o.at[pl.ds(base + (c - 1) * CH, CH)], wsem)
        last = (NCH - 1) % 2
        h_g[last].wait()
        h_w[last] = pltpu.async_copy(
            fb.at[last], fi_o.at[pl.ds(base + (NCH - 1) * CH, CH)], wsem)
        h_w[1 - last].wait()
        h_w[last].wait()

    return k(item2, F)


def _sc_gather_pairs(user2, item2, GuP, GiP, LP):
    @functools.partial(
        pl.kernel, mesh=_MESH,
        out_type=[
            jax.ShapeDtypeStruct((B, 2 * FACT), jnp.float32),
            jax.ShapeDtypeStruct((B, 2 * FACT), jnp.float32),
            jax.ShapeDtypeStruct((B, 2 * FACT), jnp.float32),
        ],
        scratch_types=[
            pltpu.VMEM((NCH, CH), jnp.int32),
            pltpu.VMEM((NCH, CH), jnp.int32),
            pltpu.VMEM((2, CH, 2 * FACT), jnp.float32),
            pltpu.VMEM((2, CH, 2 * FACT), jnp.float32),
            pltpu.VMEM((2, CH, 2 * FACT), jnp.float32),
            pltpu.SemaphoreType.DMA,
            pltpu.SemaphoreType.DMA,
        ],
    )
    def k(user_h, item_h, gu_h, gi_h, l_h,
          gu_o, gi_o, li_o, uidx, iidx, gub, gib, lb, gsem, wsem):
        wid = lax.axis_index("s") * NC + lax.axis_index("c")
        base = wid * BPW
        pltpu.sync_copy(user_h.at[wid], uidx)
        pltpu.sync_copy(item_h.at[wid], iidx)
        bufs = (gub, gib, lb)
        h_g = [None, None]
        h_w = [None, None]
        for c in range(NCH):
            s = c % 2
            if h_w[s] is not None:
                for h in h_w[s]:
                    h.wait()
            h_g[s] = (
                pltpu.async_copy(gu_h.at[uidx.at[c]], gub.at[s], gsem),
                pltpu.async_copy(gi_h.at[iidx.at[c]], gib.at[s], gsem),
                pltpu.async_copy(l_h.at[iidx.at[c]], lb.at[s], gsem),
            )
            if c > 0:
                p = 1 - s
                off = base + (c - 1) * CH
                for h in h_g[p]:
                    h.wait()
                h_w[p] = tuple(
                    pltpu.async_copy(bf.at[p], o.at[pl.ds(off, CH)], wsem)
                    for bf, o in zip(bufs, (gu_o, gi_o, li_o)))
        last = (NCH - 1) % 2
        off = base + (NCH - 1) * CH
        for h in h_g[last]:
            h.wait()
        h_w[last] = tuple(
            pltpu.async_copy(bf.at[last], o.at[pl.ds(off, CH)], wsem)
            for bf, o in zip(bufs, (gu_o, gi_o, li_o)))
        for h in h_w[1 - last]:
            h.wait()
        for h in h_w[last]:
            h.wait()

    return k(user2, item2, GuP, GiP, LP)


def _tc_combine(feature_i, gup, gip, lp, E, mu, mi):
    BB = 2048

    def body(fe, gup_r, gip_r, lp_r, e, mu_r, mi_r, xout, guo, gio, lio):
        mum = mu_r[...] != 0
        mim = mi_r[...] != 0
        gu = jnp.where(mum, gup_r[:, FACT:2 * FACT], gup_r[:, 0:FACT])
        gi = jnp.where(mim, gip_r[:, FACT:2 * FACT], gip_r[:, 0:FACT])
        li = jnp.where(mim, lp_r[:, FACT:2 * FACT], lp_r[:, 0:FACT])
        t = jnp.dot(fe[...], e[...], preferred_element_type=jnp.float32)
        xout[...] = jnp.sum(gu * (t - li + gi), axis=1, keepdims=True)
        guo[...] = gu
        gio[...] = gi
        lio[...] = li

    spec128 = pl.BlockSpec((BB, 2 * FACT), lambda i: (i, 0))
    spec64 = pl.BlockSpec((BB, FACT), lambda i: (i, 0))
    spec1 = pl.BlockSpec((BB, 1), lambda i: (i, 0))
    return pl.pallas_call(
        body,
        grid=(B // BB,),
        in_specs=[
            pl.BlockSpec((BB, IMGF), lambda i: (i, 0)),
            spec128, spec128, spec128,
            pl.BlockSpec((IMGF, FACT), lambda i: (0, 0)),
            spec1, spec1,
        ],
        out_specs=[spec1, spec64, spec64, spec64],
        out_shape=[
            jax.ShapeDtypeStruct((B, 1), jnp.float32),
            jax.ShapeDtypeStruct((B, FACT), jnp.float32),
            jax.ShapeDtypeStruct((B, FACT), jnp.float32),
            jax.ShapeDtypeStruct((B, FACT), jnp.float32),
        ],
    )(feature_i, gup, gip, lp, E, mu, mi)


def kernel(user, item, Gu, Gi, L, E, F):
    item2 = item.reshape(NW, NCH, CH)
    userP = (user % HPU).reshape(NW, NCH, CH)
    itemP = (item % HPI).reshape(NW, NCH, CH)
    mu = (user >= HPU).astype(jnp.int32).reshape(B, 1)
    mi = (item >= HPI).astype(jnp.int32).reshape(B, 1)
    feature_i = _sc_gather_f(item2, F)
    GuP = _tc_pair_transpose(jnp.swapaxes(Gu, 0, 1), HPU // BBC)
    GiP = _tc_pair_transpose(jnp.swapaxes(Gi, 0, 1), HPI // BBC)
    LP = _tc_pair_transpose(jnp.swapaxes(L, 0, 1), HPI // BBC)
    gup, gip, lp = _sc_gather_pairs(userP, itemP, GuP, GiP, LP)
    xui, gamma_u, gamma_i, l_i = _tc_combine(feature_i, gup, gip, lp, E, mu, mi)
    return (xui.reshape(B), gamma_u, gamma_i, feature_i, l_i)
